# baseline (speedup 1.0000x reference)
import functools

import jax
import jax.numpy as jnp
from jax import lax
from jax.experimental import pallas as pl
from jax.experimental.pallas import tpu as pltpu
from jax.experimental.pallas import tpu_sc as plsc

_NC = 2
_NS = 16
_NW = _NC * _NS
_CHUNK = 32768
_LANES = 16


def _sc_mul_body(x1_hbm, x2_hbm, o_hbm, a_v, b_v, o_v):
    wid = lax.axis_index("s") * _NC + lax.axis_index("c")
    per_w = x1_hbm.shape[0] // _NW
    base = wid * per_w

    def chunk_body(i, _):
        off = base + i * _CHUNK
        pltpu.sync_copy(x1_hbm.at[pl.ds(off, _CHUNK)], a_v)
        pltpu.sync_copy(x2_hbm.at[pl.ds(off, _CHUNK)], b_v)

        def inner(j, _):
            s = j * _LANES
            o_v[pl.ds(s, _LANES)] = a_v[pl.ds(s, _LANES)] * b_v[pl.ds(s, _LANES)]
            return 0

        lax.fori_loop(0, _CHUNK // _LANES, inner, 0)
        pltpu.sync_copy(o_v, o_hbm.at[pl.ds(off, _CHUNK)])
        return 0

    lax.fori_loop(0, per_w // _CHUNK, chunk_body, 0)


def _sc_mul(x1f, x2f):
    E = x1f.shape[0]
    mesh = plsc.VectorSubcoreMesh(core_axis_name="c", subcore_axis_name="s")
    f = functools.partial(
        pl.kernel,
        out_type=jax.ShapeDtypeStruct((E,), jnp.float32),
        mesh=mesh,
        scratch_types=[
            pltpu.VMEM((_CHUNK,), jnp.float32),
            pltpu.VMEM((_CHUNK,), jnp.float32),
            pltpu.VMEM((_CHUNK,), jnp.float32),
        ],
    )(_sc_mul_body)
    return f(x1f, x2f)


def kernel(x1, x2):
    B, M, N = x1.shape
    E = B * M * N
    out = _sc_mul(x1.reshape(E), x2.reshape(E))
    return out.reshape(B, M, N)


# speedup vs baseline: 1.6333x; 1.6333x over previous
import functools

import jax
import jax.numpy as jnp
from jax import lax
from jax.experimental import pallas as pl
from jax.experimental.pallas import tpu as pltpu
from jax.experimental.pallas import tpu_sc as plsc

_NC = 2
_NS = 16
_NW = _NC * _NS
_CHUNK = 16384
_LANES = 16


def _sc_mul_body(x1_hbm, x2_hbm, o_hbm,
                 a0, b0, o0, a1, b1, o1, ls0, ls1, ss0, ss1):
    wid = lax.axis_index("s") * _NC + lax.axis_index("c")
    per_w = x1_hbm.shape[0] // _NW
    base = wid * per_w
    n = per_w // _CHUNK

    bufs = ((a0, b0, o0, ls0, ss0), (a1, b1, o1, ls1, ss1))

    def start_load(i, a, b, ls):
        off = base + i * _CHUNK
        pltpu.async_copy(x1_hbm.at[pl.ds(off, _CHUNK)], a, ls)
        pltpu.async_copy(x2_hbm.at[pl.ds(off, _CHUNK)], b, ls)

    start_load(0, a0, b0, ls0)

    def outer(g, _):
        i0 = g * 2
        for k in range(2):
            i = i0 + k
            a, b, o, ls, ss = bufs[k]
            an, bn, on, lsn, ssn = bufs[1 - k]
            nxt = i + 1

            @pl.when(nxt < n)
            def _():
                start_load(nxt, an, bn, lsn)

            off = base + i * _CHUNK
            pltpu.make_async_copy(x1_hbm.at[pl.ds(off, _CHUNK)], a, ls).wait()
            pltpu.make_async_copy(x2_hbm.at[pl.ds(off, _CHUNK)], b, ls).wait()

            @pl.when(i >= 2)
            def _():
                poff = base + (i - 2) * _CHUNK
                pltpu.make_async_copy(o, o_hbm.at[pl.ds(poff, _CHUNK)], ss).wait()

            @plsc.parallel_loop(0, _CHUNK // _LANES, unroll=8)
            def _(j):
                s = j * _LANES
                o[pl.ds(s, _LANES)] = a[pl.ds(s, _LANES)] * b[pl.ds(s, _LANES)]

            pltpu.async_copy(o, o_hbm.at[pl.ds(off, _CHUNK)], ss)
        return 0

    lax.fori_loop(0, n // 2, outer, 0)

    for k in range(2):
        _, _, o, _, ss = bufs[k]
        off = base + (n - 2 + k) * _CHUNK
        pltpu.make_async_copy(o, o_hbm.at[pl.ds(off, _CHUNK)], ss).wait()


def _sc_mul(x1f, x2f):
    E = x1f.shape[0]
    mesh = plsc.VectorSubcoreMesh(core_axis_name="c", subcore_axis_name="s")
    f = functools.partial(
        pl.kernel,
        out_type=jax.ShapeDtypeStruct((E,), jnp.float32),
        mesh=mesh,
        scratch_types=[
            pltpu.VMEM((_CHUNK,), jnp.float32),
            pltpu.VMEM((_CHUNK,), jnp.float32),
            pltpu.VMEM((_CHUNK,), jnp.float32),
            pltpu.VMEM((_CHUNK,), jnp.float32),
            pltpu.VMEM((_CHUNK,), jnp.float32),
            pltpu.VMEM((_CHUNK,), jnp.float32),
            pltpu.SemaphoreType.DMA,
            pltpu.SemaphoreType.DMA,
            pltpu.SemaphoreType.DMA,
            pltpu.SemaphoreType.DMA,
        ],
    )(_sc_mul_body)
    return f(x1f, x2f)


def kernel(x1, x2):
    B, M, N = x1.shape
    E = B * M * N
    out = _sc_mul(x1.reshape(E), x2.reshape(E))
    return out.reshape(B, M, N)


# speedup vs baseline: 1.7758x; 1.0872x over previous
import functools

import jax
import jax.numpy as jnp
from jax import lax
from jax.experimental import pallas as pl
from jax.experimental.pallas import tpu as pltpu
from jax.experimental.pallas import tpu_sc as plsc

_NC = 2
_NS = 16
_NW = _NC * _NS
_CHUNK = 16384
_LANES = 16


def _sc_mul_body(sc_elems, x1_hbm, x2_hbm, o_hbm,
                 a0, b0, o0, a1, b1, o1, ls0, ls1, ss0, ss1):
    wid = lax.axis_index("s") * _NC + lax.axis_index("c")
    per_w = sc_elems // _NW
    base = wid * per_w
    n = per_w // _CHUNK

    bufs = ((a0, b0, o0, ls0, ss0), (a1, b1, o1, ls1, ss1))

    def start_load(i, a, b, ls):
        off = base + i * _CHUNK
        pltpu.async_copy(x1_hbm.at[pl.ds(off, _CHUNK)], a, ls)
        pltpu.async_copy(x2_hbm.at[pl.ds(off, _CHUNK)], b, ls)

    start_load(0, a0, b0, ls0)

    def outer(g, _):
        i0 = g * 2
        for k in range(2):
            i = i0 + k
            a, b, o, ls, ss = bufs[k]
            an, bn, on, lsn, ssn = bufs[1 - k]
            nxt = i + 1

            @pl.when(nxt < n)
            def _():
                start_load(nxt, an, bn, lsn)

            off = base + i * _CHUNK
            pltpu.make_async_copy(x1_hbm.at[pl.ds(off, _CHUNK)], a, ls).wait()
            pltpu.make_async_copy(x2_hbm.at[pl.ds(off, _CHUNK)], b, ls).wait()

            @pl.when(i >= 2)
            def _():
                poff = base + (i - 2) * _CHUNK
                pltpu.make_async_copy(o, o_hbm.at[pl.ds(poff, _CHUNK)], ss).wait()

            @plsc.parallel_loop(0, _CHUNK // _LANES, unroll=8)
            def _(j):
                s = j * _LANES
                o[pl.ds(s, _LANES)] = a[pl.ds(s, _LANES)] * b[pl.ds(s, _LANES)]

            pltpu.async_copy(o, o_hbm.at[pl.ds(off, _CHUNK)], ss)
        return 0

    lax.fori_loop(0, n // 2, outer, 0)

    for k in range(2):
        _, _, o, _, ss = bufs[k]
        off = base + (n - 2 + k) * _CHUNK
        pltpu.make_async_copy(o, o_hbm.at[pl.ds(off, _CHUNK)], ss).wait()


def _sc_mul(x1f, x2f, sc_elems):
    mesh = plsc.VectorSubcoreMesh(core_axis_name="c", subcore_axis_name="s")
    f = functools.partial(
        pl.kernel,
        out_type=jax.ShapeDtypeStruct((sc_elems,), jnp.float32),
        mesh=mesh,
        scratch_types=[
            pltpu.VMEM((_CHUNK,), jnp.float32),
            pltpu.VMEM((_CHUNK,), jnp.float32),
            pltpu.VMEM((_CHUNK,), jnp.float32),
            pltpu.VMEM((_CHUNK,), jnp.float32),
            pltpu.VMEM((_CHUNK,), jnp.float32),
            pltpu.VMEM((_CHUNK,), jnp.float32),
            pltpu.SemaphoreType.DMA,
            pltpu.SemaphoreType.DMA,
            pltpu.SemaphoreType.DMA,
            pltpu.SemaphoreType.DMA,
        ],
    )(functools.partial(_sc_mul_body, sc_elems))
    return f(x1f, x2f)


def _tc_mul_body(x1_ref, x2_ref, o_ref):
    o_ref[...] = x1_ref[...] * x2_ref[...]


def _tc_mul(x1f, x2f, row0):
    R, N = x1f.shape
    BS = 256
    rows = R - row0
    blk0 = row0 // BS
    return pl.pallas_call(
        _tc_mul_body,
        grid=(rows // BS,),
        in_specs=[
            pl.BlockSpec((BS, N), lambda i: (i + blk0, 0)),
            pl.BlockSpec((BS, N), lambda i: (i + blk0, 0)),
        ],
        out_specs=pl.BlockSpec((BS, N), lambda i: (i, 0)),
        out_shape=jax.ShapeDtypeStruct((rows, N), x1f.dtype),
    )(x1f, x2f)


_SC_ROWS = 3328


def kernel(x1, x2):
    B, M, N = x1.shape
    R = B * M
    x1f = x1.reshape(R, N)
    x2f = x2.reshape(R, N)
    E = R * N
    sc_out = _sc_mul(x1f.reshape(E), x2f.reshape(E), _SC_ROWS * N)
    tc_out = _tc_mul(x1f, x2f, _SC_ROWS)
    out = jnp.concatenate([sc_out.reshape(_SC_ROWS, N), tc_out], axis=0)
    return out.reshape(B, M, N)


# speedup vs baseline: 6.2669x; 3.5291x over previous
import jax
import jax.numpy as jnp
from jax.experimental import pallas as pl


def _mul_kernel(x1_ref, x2_ref, o_ref):
    o_ref[...] = x1_ref[...] * x2_ref[...]


def kernel(x1, x2):
    B, M, N = x1.shape
    R = B * M
    x1f = x1.reshape(R, N)
    x2f = x2.reshape(R, N)
    BS = 256
    out = pl.pallas_call(
        _mul_kernel,
        grid=(R // BS,),
        in_specs=[
            pl.BlockSpec((BS, N), lambda i: (i, 0)),
            pl.BlockSpec((BS, N), lambda i: (i, 0)),
        ],
        out_specs=pl.BlockSpec((BS, N), lambda i: (i, 0)),
        out_shape=jax.ShapeDtypeStruct((R, N), x1.dtype),
    )(x1f, x2f)
    return out.reshape(B, M, N)
